# Initial kernel scaffold; baseline (speedup 1.0000x reference)
import jax
import jax.numpy as jnp
from jax.experimental import pallas as pl


def kernel(coords, semantic_ids, intensity, embed_table, W, b):
    raise NotImplementedError("write your pallas kernel here")


# speedup vs baseline: 1.7291x; 1.7291x over previous
import functools

import jax
import jax.numpy as jnp
import numpy as np
from jax import lax
from jax.experimental import pallas as pl
from jax.experimental.pallas import tpu as pltpu
from jax.experimental.pallas import tpu_sc as plsc

N = 500000
NUM_SEMANTIC = 100000
DIM_SEMANTIC = 64
C_DIM = 128
NUM_FREQS = 5
MAX_FREQ_LOG2 = 4.0

NPAD = 512000
B_PER_W = NPAD // 32
CHUNK = 128
NCHUNK = B_PER_W // CHUNK

BLK = 2000


def _sc_gather_fn():
    info = plsc.get_sparse_core_info()
    nc = info.num_cores

    mesh = plsc.VectorSubcoreMesh(core_axis_name="c", subcore_axis_name="s")

    @functools.partial(
        pl.kernel,
        mesh=mesh,
        compiler_params=pltpu.CompilerParams(use_tc_tiling_on_sc=False),
        out_type=jax.ShapeDtypeStruct((NPAD, DIM_SEMANTIC), jnp.float32),
        scratch_types=[
            pltpu.VMEM((CHUNK,), jnp.int32),
            pltpu.VMEM((CHUNK, DIM_SEMANTIC), jnp.float32),
            pltpu.SemaphoreType.DMA,
        ],
    )
    def sc_gather(ids_hbm, table_hbm, out_hbm, idx_v, rows_v, sem):
        wid = lax.axis_index("s") * nc + lax.axis_index("c")
        base = wid * B_PER_W

        def body(c, carry):
            off = base + c * CHUNK
            pltpu.sync_copy(ids_hbm.at[pl.ds(off, CHUNK)], idx_v)
            pltpu.async_copy(table_hbm.at[idx_v], rows_v, sem).wait()
            pltpu.sync_copy(rows_v, out_hbm.at[pl.ds(off, CHUNK)])
            return carry

        lax.fori_loop(0, NCHUNK, body, 0)

    return sc_gather


def _pos_weights():
    m = np.zeros((3, 64), np.float32)
    ph = np.zeros((1, 64), np.float32)
    freqs = 2.0 ** np.linspace(0.0, MAX_FREQ_LOG2, NUM_FREQS)
    for i in range(NUM_FREQS):
        for s in range(2):
            for c in range(3):
                k = 6 * i + 3 * s + c
                m[c, k] = freqs[i]
                ph[0, k] = 0.0 if s == 0 else np.pi / 2.0
    return jnp.asarray(m), jnp.asarray(ph)


_SIN_C = (6.2831852819, -41.341698212, 81.605064899, -76.702152496,
          42.020491157, -14.883436519, 3.2191201543)
_INV_2PI = 0.15915493667125702


def _fast_sin(t):
    u = t * jnp.float32(_INV_2PI)
    r = u - jnp.floor(u + jnp.float32(0.5))
    r2 = r * r
    acc = jnp.float32(_SIN_C[6])
    for k in range(5, -1, -1):
        acc = acc * r2 + jnp.float32(_SIN_C[k])
    return acc * r


def _tc_body(sem_ref, coords_ref, inten_ref, m_ref, ph_ref, wsin_ref,
             wsem_ref, wraw_ref, wi_ref, b_ref, out_ref):
    hi = lax.Precision.HIGHEST
    g = coords_ref[...]
    t = jnp.dot(g, m_ref[...], preferred_element_type=jnp.float32,
                precision=hi)
    t = t + ph_ref[...]
    s = _fast_sin(t)
    acc = jnp.dot(s, wsin_ref[...], preferred_element_type=jnp.float32,
                  precision=hi)
    acc = acc + jnp.dot(sem_ref[...], wsem_ref[...],
                        preferred_element_type=jnp.float32, precision=hi)
    acc = acc + jnp.dot(g, wraw_ref[...], preferred_element_type=jnp.float32,
                        precision=hi)
    acc = acc + inten_ref[...] * wi_ref[...]
    acc = acc + b_ref[...]
    out_ref[...] = acc


def kernel(coords, semantic_ids, intensity, embed_table, W, b):
    ids_pad = jnp.pad(semantic_ids.astype(jnp.int32), (0, NPAD - N))
    sem_g = _sc_gather_fn()(ids_pad, embed_table)

    cols = W.T
    m, ph = _pos_weights()
    wsin = jnp.zeros((64, C_DIM), jnp.float32).at[:30].set(cols[3:33])
    wsem = cols[33:97]
    wraw = cols[0:3]
    wi = cols[97:98]
    b2 = b.reshape(1, C_DIM)

    grid = N // BLK
    out = pl.pallas_call(
        _tc_body,
        grid=(grid,),
        in_specs=[
            pl.BlockSpec((BLK, DIM_SEMANTIC), lambda i: (i, 0)),
            pl.BlockSpec((BLK, 3), lambda i: (i, 0)),
            pl.BlockSpec((BLK, 1), lambda i: (i, 0)),
            pl.BlockSpec((3, 64), lambda i: (0, 0)),
            pl.BlockSpec((1, 64), lambda i: (0, 0)),
            pl.BlockSpec((64, C_DIM), lambda i: (0, 0)),
            pl.BlockSpec((64, C_DIM), lambda i: (0, 0)),
            pl.BlockSpec((3, C_DIM), lambda i: (0, 0)),
            pl.BlockSpec((1, C_DIM), lambda i: (0, 0)),
            pl.BlockSpec((1, C_DIM), lambda i: (0, 0)),
        ],
        out_specs=pl.BlockSpec((BLK, C_DIM), lambda i: (i, 0)),
        out_shape=jax.ShapeDtypeStruct((N, C_DIM), jnp.float32),
    )(sem_g, coords, intensity, m, ph, wsin, wsem, wraw, wi, b2)
    return out


# speedup vs baseline: 2.6561x; 1.5361x over previous
import functools

import jax
import jax.numpy as jnp
import numpy as np
from jax import lax
from jax.experimental import pallas as pl
from jax.experimental.pallas import tpu as pltpu
from jax.experimental.pallas import tpu_sc as plsc

N = 500000
NUM_SEMANTIC = 100000
DIM_SEMANTIC = 64
C_DIM = 128
NUM_FREQS = 5
MAX_FREQ_LOG2 = 4.0

NW = 32
CHUNK = 128
NCHUNK = 128
B_PER_W = CHUNK * NCHUNK
NPAD = NW * B_PER_W
NBUF = 4

BLK = 4000


def _sc_gather_fn():
    info = plsc.get_sparse_core_info()
    nc = info.num_cores

    mesh = plsc.VectorSubcoreMesh(core_axis_name="c", subcore_axis_name="s")

    @functools.partial(
        pl.kernel,
        mesh=mesh,
        compiler_params=pltpu.CompilerParams(use_tc_tiling_on_sc=False),
        out_type=jax.ShapeDtypeStruct((NPAD, DIM_SEMANTIC), jnp.float32),
        scratch_types=[
            pltpu.VMEM((NCHUNK, CHUNK), jnp.int32),
            pltpu.VMEM((NBUF, CHUNK, DIM_SEMANTIC), jnp.float32),
            pltpu.SemaphoreType.DMA((NBUF,)),
            pltpu.SemaphoreType.DMA((NBUF,)),
        ],
    )
    def sc_gather(ids_hbm, table_hbm, out_hbm, idx_v, rows_v, gsem, wsem):
        wid = lax.axis_index("s") * nc + lax.axis_index("c")
        base = wid * B_PER_W
        pltpu.sync_copy(ids_hbm.at[wid], idx_v)

        def gather(i, r):
            pltpu.async_copy(table_hbm.at[idx_v.at[i]], rows_v.at[r],
                             gsem.at[r])

        def wb_copy(i, r):
            return pltpu.make_async_copy(
                rows_v.at[r], out_hbm.at[pl.ds(base + i * CHUNK, CHUNK)],
                wsem.at[r])

        for i in range(NBUF - 1):
            gather(i, i)

        def body(i, carry):
            r = lax.rem(i, NBUF)
            r2 = lax.rem(i + NBUF - 1, NBUF)
            pltpu.make_async_copy(rows_v.at[r],
                                  out_hbm.at[pl.ds(base, CHUNK)],
                                  gsem.at[r]).wait()
            wb_copy(i, r).start()

            @pl.when(jnp.logical_and(i >= 1, i + NBUF - 1 < NCHUNK))
            def _():
                wb_copy(i, r2).wait()

            @pl.when(i + NBUF - 1 < NCHUNK)
            def _():
                gather(i + NBUF - 1, r2)

            return carry

        lax.fori_loop(0, NCHUNK, body, 0)
        for r in range(NBUF):
            pltpu.make_async_copy(rows_v.at[r],
                                  out_hbm.at[pl.ds(base, CHUNK)],
                                  wsem.at[r]).wait()

    return sc_gather


def _pos_weights():
    m = np.zeros((3, 64), np.float32)
    ph = np.zeros((1, 64), np.float32)
    freqs = 2.0 ** np.linspace(0.0, MAX_FREQ_LOG2, NUM_FREQS)
    for i in range(NUM_FREQS):
        for s in range(2):
            for c in range(3):
                k = 6 * i + 3 * s + c
                m[c, k] = freqs[i]
                ph[0, k] = 0.0 if s == 0 else np.pi / 2.0
    return jnp.asarray(m), jnp.asarray(ph)


_SIN_C = (6.2831852819, -41.341698212, 81.605064899, -76.702152496,
          42.020491157, -14.883436519, 3.2191201543)
_INV_2PI = 0.15915493667125702


def _fast_sin(t):
    u = t * jnp.float32(_INV_2PI)
    r = u - jnp.floor(u + jnp.float32(0.5))
    r2 = r * r
    acc = jnp.float32(_SIN_C[6])
    for k in range(5, -1, -1):
        acc = acc * r2 + jnp.float32(_SIN_C[k])
    return acc * r


def _tc_body(sem_ref, coords_ref, inten_ref, m_ref, ph_ref, wsin_ref,
             wsem_ref, wraw_ref, wi_ref, b_ref, out_ref):
    g = coords_ref[...]
    t = ph_ref[...]
    for c in range(3):
        t = t + g[:, c:c + 1] * m_ref[c:c + 1, :]
    s = _fast_sin(t)
    acc = jnp.dot(s, wsin_ref[...], preferred_element_type=jnp.float32)
    acc = acc + jnp.dot(sem_ref[...], wsem_ref[...],
                        preferred_element_type=jnp.float32)
    acc = acc + jnp.dot(g, wraw_ref[...], preferred_element_type=jnp.float32)
    acc = acc + inten_ref[...] * wi_ref[...]
    acc = acc + b_ref[...]
    out_ref[...] = acc


def kernel(coords, semantic_ids, intensity, embed_table, W, b):
    ids_pad = jnp.pad(semantic_ids.astype(jnp.int32), (0, NPAD - N))
    ids3 = ids_pad.reshape(NW, NCHUNK, CHUNK)
    sem_g = _sc_gather_fn()(ids3, embed_table)

    cols = W.T
    m, ph = _pos_weights()
    wsin = jnp.zeros((64, C_DIM), jnp.float32).at[:30].set(cols[3:33])
    wsem = cols[33:97]
    wraw = cols[0:3]
    wi = cols[97:98]
    b2 = b.reshape(1, C_DIM)

    grid = N // BLK
    out = pl.pallas_call(
        _tc_body,
        grid=(grid,),
        in_specs=[
            pl.BlockSpec((BLK, DIM_SEMANTIC), lambda i: (i, 0)),
            pl.BlockSpec((BLK, 3), lambda i: (i, 0)),
            pl.BlockSpec((BLK, 1), lambda i: (i, 0)),
            pl.BlockSpec((3, 64), lambda i: (0, 0)),
            pl.BlockSpec((1, 64), lambda i: (0, 0)),
            pl.BlockSpec((64, C_DIM), lambda i: (0, 0)),
            pl.BlockSpec((64, C_DIM), lambda i: (0, 0)),
            pl.BlockSpec((3, C_DIM), lambda i: (0, 0)),
            pl.BlockSpec((1, C_DIM), lambda i: (0, 0)),
            pl.BlockSpec((1, C_DIM), lambda i: (0, 0)),
        ],
        out_specs=pl.BlockSpec((BLK, C_DIM), lambda i: (i, 0)),
        out_shape=jax.ShapeDtypeStruct((N, C_DIM), jnp.float32),
    )(sem_g, coords, intensity, m, ph, wsin, wsem, wraw, wi, b2)
    return out
